# Initial kernel scaffold; baseline (speedup 1.0000x reference)
import jax
import jax.numpy as jnp
from jax.experimental import pallas as pl


def kernel(features, edge_index, W0, b0, W1, b1, W2, b2):
    raise NotImplementedError("write your pallas kernel here")


# speedup vs baseline: 4.3390x; 4.3390x over previous
import functools

import jax
import jax.numpy as jnp
from jax import lax
from jax.experimental import pallas as pl
from jax.experimental.pallas import tpu as pltpu
from jax.experimental.pallas import tpu_sc as plsc

N = 10000
E = 160000
D_IN = 256
D_H = 512
D_OUT = 40
DC = 128
DC2 = 128
NC = 2
NS = 16
RPT = N // NS
B = 128

_mesh = functools.partial(
    plsc.VectorSubcoreMesh, core_axis_name="c", subcore_axis_name="s")


def _fill_zeros(zbuf, rows, cols):
  nz = cols // 16

  def body(r, _):
    for j in range(nz):
      zbuf[r, j * 16:(j + 1) * 16] = jnp.zeros((16,), jnp.float32)
    return 0

  lax.fori_loop(0, rows, body, 0)


def _zero_acc(zbuf, acc, r0):
  for j in range(5):
    pltpu.sync_copy(zbuf, acc.at[pl.ds(r0 + j * 125, 125)])


def _hist_call(eflat):
  @functools.partial(
      pl.kernel,
      out_type=jax.ShapeDtypeStruct((2, NS, RPT, DC), jnp.float32),
      mesh=_mesh(),
      scratch_types=[
          pltpu.VMEM((B,), jnp.int32),
          pltpu.VMEM((16,), jnp.int32),
          pltpu.VMEM((B, DC), jnp.float32),
          pltpu.VMEM((16, DC), jnp.float32),
          pltpu.VMEM((125, DC), jnp.float32),
          pltpu.VMEM_SHARED((N, DC), jnp.float32),
      ],
  )
  def k(ei_hbm, out_hbm, idx_v, idx16, onesb, ones16, zbuf, acc):
    cid = lax.axis_index("c")
    sid = lax.axis_index("s")

    def fill_ones(buf, rows):
      def body(r, _):
        for j in range(DC // 16):
          buf[r, j * 16:(j + 1) * 16] = jnp.ones((16,), jnp.float32)
        return 0
      lax.fori_loop(0, rows, body, 0)

    fill_ones(onesb, B)
    fill_ones(ones16, 16)
    _fill_zeros(zbuf, 125, DC)
    r0 = sid * RPT
    _zero_acc(zbuf, acc, r0)
    plsc.subcore_barrier()

    ept = E // NS
    nb = ept // B
    ebase = cid * E + sid * ept

    def body(t, _):
      pltpu.sync_copy(ei_hbm.at[pl.ds(ebase + t * B, B)], idx_v)
      pltpu.sync_copy(onesb, acc.at[idx_v], add=True)
      return 0

    lax.fori_loop(0, nb, body, 0)
    pltpu.sync_copy(ei_hbm.at[pl.ds(ebase + nb * B, 16)], idx16)
    pltpu.sync_copy(ones16, acc.at[idx16], add=True)
    plsc.subcore_barrier()
    pltpu.sync_copy(acc.at[pl.ds(r0, RPT)], out_hbm.at[cid, sid])

  return k(eflat)


def _agg_cols_call(src, dst, tables):
  nt = len(tables)
  cpc = nt // 2

  @functools.partial(
      pl.kernel,
      out_type=[jax.ShapeDtypeStruct((NS, RPT, DC), jnp.float32)] * nt,
      mesh=_mesh(),
      scratch_types=[
          pltpu.VMEM((B,), jnp.int32),
          pltpu.VMEM((B,), jnp.int32),
          pltpu.VMEM((16,), jnp.int32),
          pltpu.VMEM((16,), jnp.int32),
          pltpu.VMEM((B, DC), jnp.float32),
          pltpu.VMEM((16, DC), jnp.float32),
          pltpu.VMEM((125, DC), jnp.float32),
          pltpu.VMEM_SHARED((N, DC), jnp.float32),
          pltpu.SemaphoreType.DMA,
      ],
  )
  def k(src_hbm, dst_hbm, *rest):
    tabs = rest[:nt]
    outs = rest[nt:2 * nt]
    (src_v, dst_v, src16, dst16, msgs, msgs16, zbuf, acc, sem) = rest[2 * nt:]
    cid = lax.axis_index("c")
    sid = lax.axis_index("s")
    _fill_zeros(zbuf, 125, DC)
    r0 = sid * RPT

    ept = E // NS
    nb = ept // B
    ebase = sid * ept

    def scatter_pass(tab):
      def body(t, _):
        b0 = ebase + t * B
        pltpu.sync_copy(src_hbm.at[pl.ds(b0, B)], src_v)
        pltpu.sync_copy(dst_hbm.at[pl.ds(b0, B)], dst_v)
        pltpu.async_copy(tab.at[src_v], msgs, sem).wait()
        pltpu.sync_copy(msgs, acc.at[dst_v], add=True)
        return 0

      lax.fori_loop(0, nb, body, 0)
      b0 = ebase + nb * B
      pltpu.sync_copy(src_hbm.at[pl.ds(b0, 16)], src16)
      pltpu.sync_copy(dst_hbm.at[pl.ds(b0, 16)], dst16)
      pltpu.async_copy(tab.at[src16], msgs16, sem).wait()
      pltpu.sync_copy(msgs16, acc.at[dst16], add=True)

    def copy_out(out):
      pltpu.sync_copy(acc.at[pl.ds(r0, RPT)], out.at[sid])

    _zero_acc(zbuf, acc, r0)
    plsc.subcore_barrier()
    for i in range(cpc):
      @pl.when(cid == 0)
      def _():
        scatter_pass(tabs[i])

      @pl.when(cid == 1)
      def _():
        scatter_pass(tabs[cpc + i])

      plsc.subcore_barrier()

      @pl.when(cid == 0)
      def _():
        copy_out(outs[i])

      @pl.when(cid == 1)
      def _():
        copy_out(outs[cpc + i])

      if i < cpc - 1:
        _zero_acc(zbuf, acc, r0)
        plsc.subcore_barrier()

  return k(src, dst, *tables)


def _agg_edges_call(src, dst, table):
  @functools.partial(
      pl.kernel,
      out_type=jax.ShapeDtypeStruct((2, NS, RPT, DC2), jnp.float32),
      mesh=_mesh(),
      scratch_types=[
          pltpu.VMEM((B,), jnp.int32),
          pltpu.VMEM((B,), jnp.int32),
          pltpu.VMEM((8,), jnp.int32),
          pltpu.VMEM((8,), jnp.int32),
          pltpu.VMEM((B, DC2), jnp.float32),
          pltpu.VMEM((8, DC2), jnp.float32),
          pltpu.VMEM((125, DC2), jnp.float32),
          pltpu.VMEM_SHARED((N, DC2), jnp.float32),
          pltpu.SemaphoreType.DMA,
      ],
  )
  def k(src_hbm, dst_hbm, tab, out_hbm, src_v, dst_v, src8, dst8, msgs,
        msgs8, zbuf, acc, sem):
    cid = lax.axis_index("c")
    sid = lax.axis_index("s")
    _fill_zeros(zbuf, 125, DC2)
    r0 = sid * RPT
    _zero_acc(zbuf, acc, r0)
    plsc.subcore_barrier()

    ept = E // (NC * NS)
    nb = ept // B
    ebase = cid * (E // NC) + sid * ept

    def body(t, _):
      b0 = ebase + t * B
      pltpu.sync_copy(src_hbm.at[pl.ds(b0, B)], src_v)
      pltpu.sync_copy(dst_hbm.at[pl.ds(b0, B)], dst_v)
      pltpu.async_copy(tab.at[src_v], msgs, sem).wait()
      pltpu.sync_copy(msgs, acc.at[dst_v], add=True)
      return 0

    lax.fori_loop(0, nb, body, 0)
    b0 = ebase + nb * B
    pltpu.sync_copy(src_hbm.at[pl.ds(b0, 8)], src8)
    pltpu.sync_copy(dst_hbm.at[pl.ds(b0, 8)], dst8)
    pltpu.async_copy(tab.at[src8], msgs8, sem).wait()
    pltpu.sync_copy(msgs8, acc.at[dst8], add=True)
    plsc.subcore_barrier()
    pltpu.sync_copy(acc.at[pl.ds(r0, RPT)], out_hbm.at[cid, sid])

  return k(src, dst, table)


_R = 1000


def _norms(hist_blk):
  deg_o = hist_blk[0, :, 0:1]
  deg_i = hist_blk[1, :, 0:1]
  return (lax.rsqrt(jnp.maximum(deg_o, 1.0)),
          lax.rsqrt(jnp.maximum(deg_i, 1.0)))


def _ep0_call(x, hist):
  def body(x_ref, h_ref, o0_ref, o1_ref):
    no, _ = _norms(h_ref[...])
    xs = x_ref[...] * no
    o0_ref[...] = xs[:, :DC]
    o1_ref[...] = xs[:, DC:]

  return pl.pallas_call(
      body,
      grid=(N // _R,),
      in_specs=[
          pl.BlockSpec((_R, D_IN), lambda i: (i, 0)),
          pl.BlockSpec((2, _R, DC), lambda i: (0, i, 0)),
      ],
      out_specs=[pl.BlockSpec((_R, DC), lambda i: (i, 0))] * 2,
      out_shape=[jax.ShapeDtypeStruct((N, DC), jnp.float32)] * 2,
  )(x, hist)


def _mm0_call(a0, a1, hist, w0, b0):
  def body(a0_ref, a1_ref, h_ref, w_ref, b_ref, *o_refs):
    no, ni = _norms(h_ref[...])
    a = jnp.concatenate([a0_ref[...], a1_ref[...]], axis=1) * ni
    h = jnp.dot(a, w_ref[...], preferred_element_type=jnp.float32,
                precision=lax.Precision.HIGHEST)
    h = jnp.maximum(h + b_ref[...], 0.0) * no
    for j in range(4):
      o_refs[j][...] = h[:, j * DC:(j + 1) * DC]

  return pl.pallas_call(
      body,
      grid=(N // _R,),
      in_specs=[
          pl.BlockSpec((_R, DC), lambda i: (i, 0)),
          pl.BlockSpec((_R, DC), lambda i: (i, 0)),
          pl.BlockSpec((2, _R, DC), lambda i: (0, i, 0)),
          pl.BlockSpec((D_IN, D_H), lambda i: (0, 0)),
          pl.BlockSpec((1, D_H), lambda i: (0, 0)),
      ],
      out_specs=[pl.BlockSpec((_R, DC), lambda i: (i, 0))] * 4,
      out_shape=[jax.ShapeDtypeStruct((N, DC), jnp.float32)] * 4,
  )(a0, a1, hist, w0, b0)


def _mm12_call(aggs, hist, w1, b1, w2p):
  def body(a0_ref, a1_ref, a2_ref, a3_ref, h_ref, w1_ref, b1_ref, w2_ref,
           o_ref):
    no, ni = _norms(h_ref[...])
    a = jnp.concatenate(
        [a0_ref[...], a1_ref[...], a2_ref[...], a3_ref[...]], axis=1) * ni
    t = jnp.dot(a, w1_ref[...], preferred_element_type=jnp.float32,
                precision=lax.Precision.HIGHEST)
    t = jnp.maximum(t + b1_ref[...], 0.0) * no
    o_ref[...] = jnp.dot(t, w2_ref[...], preferred_element_type=jnp.float32,
                         precision=lax.Precision.HIGHEST)

  return pl.pallas_call(
      body,
      grid=(N // _R,),
      in_specs=[pl.BlockSpec((_R, DC), lambda i: (i, 0))] * 4 + [
          pl.BlockSpec((2, _R, DC), lambda i: (0, i, 0)),
          pl.BlockSpec((D_H, D_H), lambda i: (0, 0)),
          pl.BlockSpec((1, D_H), lambda i: (0, 0)),
          pl.BlockSpec((D_H, DC2), lambda i: (0, 0)),
      ],
      out_specs=pl.BlockSpec((_R, DC2), lambda i: (i, 0)),
      out_shape=jax.ShapeDtypeStruct((N, DC2), jnp.float32),
  )(*aggs, hist, w1, b1, w2p)


def _final_call(p, hist, b2):
  def body(p_ref, h_ref, b_ref, o_ref):
    _, ni = _norms(h_ref[...])
    s = (p_ref[0] + p_ref[1])[:, :D_OUT]
    o_ref[...] = s * ni + b_ref[...]

  return pl.pallas_call(
      body,
      grid=(N // _R,),
      in_specs=[
          pl.BlockSpec((2, _R, DC2), lambda i: (0, i, 0)),
          pl.BlockSpec((2, _R, DC), lambda i: (0, i, 0)),
          pl.BlockSpec((1, D_OUT), lambda i: (0, 0)),
      ],
      out_specs=pl.BlockSpec((_R, D_OUT), lambda i: (i, 0)),
      out_shape=jax.ShapeDtypeStruct((N, D_OUT), jnp.float32),
  )(p, hist, b2)


def kernel(features, edge_index, W0, b0, W1, b1, W2, b2):
  ei = edge_index.astype(jnp.int32)
  src = ei[0]
  dst = ei[1]
  hist = _hist_call(jnp.concatenate([src, dst])).reshape(2, N, DC)

  xs0, xs1 = _ep0_call(features, hist)
  a00, a01 = _agg_cols_call(src, dst, [xs0, xs1])
  h1 = _mm0_call(a00.reshape(N, DC), a01.reshape(N, DC), hist, W0,
                 b0.reshape(1, D_H))

  a1 = _agg_cols_call(src, dst, list(h1))
  a1 = [a.reshape(N, DC) for a in a1]
  w2p = jnp.concatenate(
      [W2, jnp.zeros((D_H, DC2 - D_OUT), jnp.float32)], axis=1)
  m2 = _mm12_call(a1, hist, W1, b1.reshape(1, D_H), w2p)

  p = _agg_edges_call(src, dst, m2).reshape(2, N, DC2)
  return _final_call(p, hist, b2.reshape(1, D_OUT))
